# Initial kernel scaffold; baseline (speedup 1.0000x reference)
import jax
import jax.numpy as jnp
from jax.experimental import pallas as pl


def kernel(pos, batch, W_ffm, b_ffm, W_nnx, b_nnx, bn1_g, bn1_b, lin1_W, lin1_b, bn2_g, bn2_b, lin2_W, lin2_b, rc, gp_bn_g, gp_bn_b, gp_W, gp_b, reg_W1, reg_b1, reg_W2, reg_b2, reg_W3, reg_b3):
    raise NotImplementedError("write your pallas kernel here")


# speedup vs baseline: 3.9665x; 3.9665x over previous
import functools

import jax
import jax.numpy as jnp
from jax.experimental import pallas as pl
from jax.experimental.pallas import tpu as pltpu

N = 100000
NG = 32
H = 128
L = 4
GP = 256
M1, M2 = 256, 128
NC = 64

BLK = 2000
NB = N // BLK

NEG_INF = float("-inf")


def _encoder_body(
    pos_ref, batch_ref,
    W_ffm_ref, b_ffm_ref,
    W_nnx_ref, b_nnx_ref,
    s1_ref, b1_ref,
    l1W_ref, l1b_ref,
    l2W_ref, l2b_ref,
    gps_ref, gpb2_ref, gpW_ref, gpb_ref,
    rW1_ref, rb1_ref, rW2_ref, rb2_ref, rW3_ref, rb3_ref,
    out_ref, acc_ref,
):
    i = pl.program_id(0)

    @pl.when(i == 0)
    def _init():
        acc_ref[...] = jnp.full((NG, GP), NEG_INF, jnp.float32)

    x = jnp.dot(pos_ref[...], W_ffm_ref[...],
                preferred_element_type=jnp.float32) + b_ffm_ref[...]

    t = jnp.maximum(
        jnp.dot(x, W_nnx_ref[...], preferred_element_type=jnp.float32)
        + b_nnx_ref[...], 0.0)
    t = jnp.maximum(
        jnp.dot(t, l1W_ref[0], preferred_element_type=jnp.float32)
        + l1b_ref[0:1, :], 0.0)
    x = x + jnp.dot(t, l2W_ref[0], preferred_element_type=jnp.float32) \
        + l2b_ref[0:1, :]

    for il in range(1, L):
        t = jnp.maximum(x * s1_ref[il:il + 1, :] + b1_ref[il:il + 1, :], 0.0)
        t = jnp.maximum(
            jnp.dot(t, l1W_ref[il], preferred_element_type=jnp.float32)
            + l1b_ref[il:il + 1, :], 0.0)
        x = x + jnp.dot(t, l2W_ref[il], preferred_element_type=jnp.float32) \
            + l2b_ref[il:il + 1, :]

    g = jnp.dot(jnp.maximum(x * gps_ref[...] + gpb2_ref[...], 0.0),
                gpW_ref[...], preferred_element_type=jnp.float32) \
        + gpb_ref[...]

    b_col = batch_ref[...]
    s_lo = batch_ref[0, 0]
    s_hi = batch_ref[BLK - 1, 0]
    seg_ids = jax.lax.broadcasted_iota(jnp.int32, (NG, 1), 0)

    def seg_body(s, _):
        part = jnp.max(jnp.where(b_col == s, g, NEG_INF), axis=0,
                       keepdims=True)
        acc = acc_ref[...]
        acc_ref[...] = jnp.where(seg_ids == s,
                                 jnp.maximum(acc, part), acc)
        return 0

    jax.lax.fori_loop(s_lo, s_hi + 1, seg_body, 0)

    @pl.when(i == NB - 1)
    def _regress():
        p = acc_ref[...]
        h = jnp.maximum(
            jnp.dot(p, rW1_ref[...], preferred_element_type=jnp.float32)
            + rb1_ref[...], 0.0)
        h = jnp.maximum(
            jnp.dot(h, rW2_ref[...], preferred_element_type=jnp.float32)
            + rb2_ref[...], 0.0)
        out_ref[...] = jnp.dot(h, rW3_ref[...],
                               preferred_element_type=jnp.float32) \
            + rb3_ref[...]


def kernel(pos, batch, W_ffm, b_ffm, W_nnx, b_nnx, bn1_g, bn1_b, lin1_W,
           lin1_b, bn2_g, bn2_b, lin2_W, lin2_b, rc, gp_bn_g, gp_bn_b,
           gp_W, gp_b, reg_W1, reg_b1, reg_W2, reg_b2, reg_W3, reg_b3):
    inv = 1.0 / jnp.sqrt(1.0 + 1e-5)
    s1 = bn1_g * inv
    s2 = bn2_g * inv
    W_nnx_f = W_nnx * s1[0][None, :]
    b_nnx_f = (b_nnx * s1[0] + bn1_b[0]).reshape(1, H)
    lin1_Wf = lin1_W * s2[:, None, :]
    lin1_bf = lin1_b * s2 + bn2_b
    lin2_Wf = lin2_W * rc[:, None, None]
    lin2_bf = lin2_b * rc[:, None]
    gps = (gp_bn_g * inv).reshape(1, H)
    gpb2 = gp_bn_b.reshape(1, H)

    full = lambda shape: pl.BlockSpec(shape, lambda i: tuple(0 for _ in shape))

    out = pl.pallas_call(
        _encoder_body,
        grid=(NB,),
        in_specs=[
            pl.BlockSpec((BLK, 3), lambda i: (i, 0)),
            pl.BlockSpec((BLK, 1), lambda i: (i, 0)),
            full((3, H)), full((1, H)),
            full((H, H)), full((1, H)),
            full((L, H)), full((L, H)),
            full((L, H, H)), full((L, H)),
            full((L, H, H)), full((L, H)),
            full((1, H)), full((1, H)), full((H, GP)), full((1, GP)),
            full((GP, M1)), full((1, M1)),
            full((M1, M2)), full((1, M2)),
            full((M2, NC)), full((1, NC)),
        ],
        out_specs=pl.BlockSpec((NG, NC), lambda i: (0, 0)),
        out_shape=jax.ShapeDtypeStruct((NG, NC), jnp.float32),
        scratch_shapes=[pltpu.VMEM((NG, GP), jnp.float32)],
    )(
        pos, batch.reshape(N, 1),
        W_ffm, b_ffm.reshape(1, H),
        W_nnx_f, b_nnx_f,
        s1, bn1_b,
        lin1_Wf, lin1_bf,
        lin2_Wf, lin2_bf,
        gps, gpb2, gp_W, gp_b.reshape(1, GP),
        reg_W1, reg_b1.reshape(1, M1),
        reg_W2, reg_b2.reshape(1, M2),
        reg_W3, reg_b3.reshape(1, NC),
    )
    return out
